# baseline (speedup 1.0000x reference)
import functools

import jax
import jax.numpy as jnp
from jax import lax
from jax.experimental import pallas as pl
from jax.experimental.pallas import tpu as pltpu
from jax.experimental.pallas import tpu_sc as plsc

D = 64
CHUNK = 200
SPLIT = 128
NBUF = 2
SCALE = 8.0


def _scale_chunk(src, dst):
    @plsc.parallel_loop(0, CHUNK, step=1, unroll=8)
    def _(i):
        for k in range(D // 16):
            sl = pl.ds(k * 16, 16)
            dst[i, sl] = src[i, sl] * SCALE


def _make_emb_call(bsz, seq):
    assert seq == CHUNK
    info = plsc.get_sparse_core_info()
    nw = info.num_cores * info.num_subcores
    assert bsz % nw == 0
    steps = bsz // nw
    groups = steps // NBUF
    assert groups >= 3 and steps % NBUF == 0

    mesh = plsc.VectorSubcoreMesh(core_axis_name="c", subcore_axis_name="s")

    @functools.partial(
        pl.kernel,
        out_type=jax.ShapeDtypeStruct((bsz, seq, D), jnp.float32),
        mesh=mesh,
        compiler_params=pltpu.CompilerParams(use_tc_tiling_on_sc=False),
        scratch_types=[
            pltpu.VMEM((steps, CHUNK), jnp.int32),
            [pltpu.VMEM((CHUNK, D), jnp.float32) for _ in range(NBUF)],
            [pltpu.VMEM((CHUNK, D), jnp.float32) for _ in range(NBUF)],
            [pltpu.SemaphoreType.DMA for _ in range(NBUF)],
            [pltpu.SemaphoreType.DMA for _ in range(NBUF)],
        ],
    )
    def emb(idx_hbm, table_hbm, out_hbm, idx_v, gbufs, sbufs, gsems, ssems):
        wid = lax.axis_index("s") * info.num_cores + lax.axis_index("c")
        b0 = wid * steps

        pltpu.sync_copy(idx_hbm.at[pl.ds(b0, steps)], idx_v)

        def gather_copies(b, s):
            return (
                pltpu.make_async_copy(
                    table_hbm.at[idx_v.at[s, pl.ds(0, SPLIT)]],
                    gbufs[b].at[pl.ds(0, SPLIT)],
                    gsems[b],
                ),
                pltpu.make_async_copy(
                    table_hbm.at[idx_v.at[s, pl.ds(SPLIT, CHUNK - SPLIT)]],
                    gbufs[b].at[pl.ds(SPLIT, CHUNK - SPLIT)],
                    gsems[b],
                ),
            )

        def start_gather(b, s):
            for c in gather_copies(b, s):
                c.start()

        def wait_gather(b, s):
            for c in gather_copies(b, s):
                c.wait()

        def store_copy(b, s):
            return pltpu.make_async_copy(sbufs[b], out_hbm.at[b0 + s], ssems[b])

        for b in range(NBUF):
            start_gather(b, b)

        for b in range(NBUF):
            wait_gather(b, b)
            _scale_chunk(gbufs[b], sbufs[b])
            store_copy(b, b).start()
            start_gather(b, NBUF + b)

        def group(g, _):
            for b in range(NBUF):
                s = g * NBUF + b
                wait_gather(b, s)
                store_copy(b, s - NBUF).wait()
                _scale_chunk(gbufs[b], sbufs[b])
                store_copy(b, s).start()
                start_gather(b, s + NBUF)
            return 0

        lax.fori_loop(1, groups - 1, group, 0)

        for b in range(NBUF):
            s = (groups - 1) * NBUF + b
            wait_gather(b, s)
            store_copy(b, s - NBUF).wait()
            _scale_chunk(gbufs[b], sbufs[b])
            store_copy(b, s).start()
        for b in range(NBUF):
            store_copy(b, (groups - 1) * NBUF + b).wait()

    return emb


def kernel(inp_tokens, emb_table):
    bsz, seq = inp_tokens.shape
    return _make_emb_call(bsz, seq)(inp_tokens, emb_table)
